# baseline (speedup 1.0000x reference)
import numpy as np
import jax
import jax.numpy as jnp
from jax.experimental import pallas as pl
from jax.experimental.pallas import tpu as pltpu

_REMAP = np.array([1, 2, 3, 4, 5, 6, 7, 8, 9, 10, 11, 13, 14, 15, 16, 17,
                   18, 19, 20, 21, 22, 23, 24, 25, 27, 28, 31, 32, 33, 34,
                   35, 36, 37, 38, 39, 40, 41, 42, 43, 44, 46, 47, 48, 49,
                   50, 51, 52, 53, 54, 55, 56, 57, 58, 59, 60, 61, 62, 63,
                   64, 65, 67, 70, 72, 73, 74, 75, 76, 77, 78, 79, 80, 81,
                   82, 84, 85, 86, 87, 88, 89, 90], dtype=np.int32)

_KEEP = np.zeros((91,), dtype=bool)
_KEEP[_REMAP] = True
_R = np.zeros((91, 80), dtype=np.float32)
_R[_REMAP, np.arange(80)] = 1.0

_Q = 900
_QP = 1024
_CH = 128
_K = 300
_C = 91


def _body(off_ref, keep_ref, r_ref, lg_ref, bx_ref, outl_ref, outb_ref):
    lg = lg_ref[0]
    bx = bx_ref[0]

    keep = keep_ref[...] > 0
    masked = jnp.where(keep, lg, -jnp.inf)
    vcol = jnp.max(masked, axis=1, keepdims=True)
    vcol = jnp.concatenate(
        [vcol, jnp.full((_QP - _Q, 1), -jnp.inf, jnp.float32)], axis=0)
    vrow = vcol.reshape(1, _QP)

    jj = jax.lax.broadcasted_iota(jnp.int32, (_CH, _QP), 1)
    colranks = jnp.zeros((1, _QP), jnp.int32)
    for c in range(_QP // _CH):
        vic = vcol[c * _CH:(c + 1) * _CH]
        ic = jax.lax.broadcasted_iota(jnp.int32, (_CH, _QP), 0) + (c * _CH)
        beats = (vic > vrow) | ((vic == vrow) & (ic < jj))
        colranks = colranks + jnp.sum(beats.astype(jnp.int32), axis=0,
                                      keepdims=True)

    riota = jax.lax.broadcasted_iota(jnp.int32, (_K, _QP), 0)
    jiota = jax.lax.broadcasted_iota(jnp.int32, (_K, _QP), 1)
    sel = (riota == colranks).astype(jnp.float32)
    ind = jnp.sum(sel * jiota.astype(jnp.float32), axis=1,
                  keepdims=True)
    idx = ind + off_ref[0]

    sel2 = (jiota.astype(jnp.float32) == idx)[:, :_Q].astype(jnp.float32)

    gl = jnp.dot(sel2, lg, preferred_element_type=jnp.float32, precision=jax.lax.Precision.HIGHEST)
    outl_ref[0] = jnp.dot(gl, r_ref[...],
                          preferred_element_type=jnp.float32, precision=jax.lax.Precision.HIGHEST)
    outb_ref[0] = jnp.dot(sel2, bx, preferred_element_type=jnp.float32, precision=jax.lax.Precision.HIGHEST)


def kernel(pred_logits, pred_boxes, student_num_queries):
    bs = pred_logits.shape[0]
    off = (jnp.asarray(student_num_queries, jnp.int32) - _K).astype(jnp.float32)
    off = off.reshape(1)
    grid_spec = pl.GridSpec(
        grid=(bs,),
        in_specs=[
            pl.BlockSpec(memory_space=pltpu.SMEM),
            pl.BlockSpec((1, _C), lambda b: (0, 0)),
            pl.BlockSpec((_C, 80), lambda b: (0, 0)),
            pl.BlockSpec((1, _Q, _C), lambda b: (b, 0, 0)),
            pl.BlockSpec((1, _Q, 4), lambda b: (b, 0, 0)),
        ],
        out_specs=[
            pl.BlockSpec((1, _K, 80), lambda b: (b, 0, 0)),
            pl.BlockSpec((1, _K, 4), lambda b: (b, 0, 0)),
        ],
    )
    return pl.pallas_call(
        _body,
        grid_spec=grid_spec,
        out_shape=[
            jax.ShapeDtypeStruct((bs, _K, 80), jnp.float32),
            jax.ShapeDtypeStruct((bs, _K, 4), jnp.float32),
        ],
    )(off, jnp.asarray(_KEEP, jnp.float32).reshape(1, _C),
      jnp.asarray(_R), pred_logits, pred_boxes)
